# baseline (speedup 1.0000x reference)
import functools

import jax
import jax.numpy as jnp
from jax import lax
from jax.experimental import pallas as pl
from jax.experimental.pallas import tpu as pltpu
from jax.experimental.pallas import tpu_sc as plsc

N = 10000
E = 320000
NC = 2
NT = 16
NW = NC * NT
R = 1000

PH = 5120
ACCR = 5248
TPT = ACCR // NT
DUMP = 5200
TPAD = 10496

NP = 10240
ED_MAIN = 10112
ED_LAST = E - (NW - 1) * ED_MAIN
BCAP = 10240
BROWS = BCAP // 128


def _mesh():
    return plsc.VectorSubcoreMesh(
        core_axis_name="c", subcore_axis_name="s", num_cores=NC, num_subcores=NT
    )


@functools.partial(
    pl.kernel,
    out_type=[
        jax.ShapeDtypeStruct((NW, 2, BCAP), jnp.int32),
        jax.ShapeDtypeStruct((NW, 2, BCAP), jnp.int32),
        jax.ShapeDtypeStruct((NW, 2, BROWS, 128), jnp.int32),
        jax.ShapeDtypeStruct((NW, 128), jnp.int32),
        jax.ShapeDtypeStruct((NW, NP), jnp.float32),
    ],
    mesh=_mesh(),
    scratch_types=[
        pltpu.VMEM((ED_MAIN,), jnp.int32),
        pltpu.VMEM((ED_MAIN,), jnp.int32),
        pltpu.VMEM((NP,), jnp.float32),
        pltpu.VMEM((BCAP,), jnp.int32),
        pltpu.VMEM((BCAP,), jnp.int32),
        pltpu.VMEM((BCAP,), jnp.int32),
        pltpu.VMEM((BCAP,), jnp.int32),
        pltpu.VMEM((BROWS, 128), jnp.int32),
        pltpu.VMEM((BROWS, 128), jnp.int32),
        pltpu.VMEM((128,), jnp.int32),
    ],
    compiler_params=pltpu.CompilerParams(needs_layout_passes=False),
)
def _bucket_kernel(
    srcflat, dstflat, bsrc, bsrcT, bdst, bcnt, bhist,
    in_s, in_d, hist, b0s, b1s, b0t, b1t, b0d, b1d, cnt_v
):
    c = lax.axis_index("c")
    s = lax.axis_index("s")
    w = c * NT + s
    zeros_f = jnp.zeros((16,), jnp.float32)
    zero_i = jnp.zeros((16,), jnp.int32)
    ones_f = jnp.full((16,), 1.0, jnp.float32)
    dump_i = jnp.full((16,), DUMP, jnp.int32)
    iota = lax.iota(jnp.int32, 16)

    def zstep(i, carry):
        off = pl.ds(pl.multiple_of(16 * i, 16), 16)
        hist[off] = zeros_f
        return carry

    lax.fori_loop(0, NP // 16, zstep, 0)

    def pstep(i, carry):
        off = pl.ds(pl.multiple_of(16 * i, 16), 16)
        b0s[off] = zero_i
        b1s[off] = zero_i
        b0t[off] = zero_i + TPAD
        b1t[off] = zero_i + TPAD
        pos = iota + 16 * i
        rowi = pos >> 7
        coli = pos & 127
        plsc.store_scatter(b0d, [rowi, coli], dump_i)
        plsc.store_scatter(b1d, [rowi, coli], dump_i)
        return carry

    lax.fori_loop(0, BCAP // 16, pstep, 0)

    @pl.when(w < NW - 1)
    def _():
        pltpu.sync_copy(srcflat.at[pl.ds(ED_MAIN * w, ED_MAIN)], in_s)
        pltpu.sync_copy(dstflat.at[pl.ds(ED_MAIN * w, ED_MAIN)], in_d)

    @pl.when(w == NW - 1)
    def _():
        pltpu.sync_copy(
            srcflat.at[pl.ds(ED_MAIN * w, ED_LAST)], in_s.at[pl.ds(0, ED_LAST)]
        )
        pltpu.sync_copy(
            dstflat.at[pl.ds(ED_MAIN * w, ED_LAST)], in_d.at[pl.ds(0, ED_LAST)]
        )

    nvec = jnp.where(w == NW - 1, ED_LAST // 16, ED_MAIN // 16)

    def step(i, carry):
        off0, off1 = carry
        sl = pl.ds(pl.multiple_of(16 * i, 16), 16)
        sv = in_s[sl]
        dv = in_d[sl]
        plsc.addupdate_scatter(hist, [dv], ones_f)
        m0 = dv < PH
        m1 = jnp.logical_not(m0)
        r0 = plsc.cumsum(m0.astype(jnp.int32))
        r1 = plsc.cumsum(m1.astype(jnp.int32))
        pos0 = off0 + r0 - 1
        pos1 = off1 + r1 - 1
        plsc.store_scatter(b0s, [pos0], sv, mask=m0)
        plsc.store_scatter(b1s, [pos1], sv, mask=m1)
        plsc.store_scatter(b0t, [pos0], sv + TPAD, mask=m0)
        plsc.store_scatter(b1t, [pos1], sv + TPAD, mask=m1)
        plsc.store_scatter(b0d, [pos0 >> 7, pos0 & 127], dv, mask=m0)
        plsc.store_scatter(b1d, [pos1 >> 7, pos1 & 127], dv - PH, mask=m1)
        return (off0 + jnp.max(r0), off1 + jnp.max(r1))

    off0, off1 = lax.fori_loop(0, nvec, step, (jnp.int32(0), jnp.int32(0)))

    for i in range(8):
        cnt_v[pl.ds(16 * i, 16)] = zero_i
    cnt_v[pl.ds(0, 16)] = off0 * (iota == 0) + off1 * (iota == 1)

    pltpu.sync_copy(hist, bhist.at[w])
    pltpu.sync_copy(cnt_v, bcnt.at[w])
    pltpu.sync_copy(b0s, bsrc.at[w].at[0])
    pltpu.sync_copy(b1s, bsrc.at[w].at[1])
    pltpu.sync_copy(b0t, bsrcT.at[w].at[0])
    pltpu.sync_copy(b1t, bsrcT.at[w].at[1])
    pltpu.sync_copy(b0d, bdst.at[w].at[0])
    pltpu.sync_copy(b1d, bdst.at[w].at[1])


_SCAT_SCRATCH = [
    pltpu.VMEM((2 * BCAP + 128,), jnp.int32),
    pltpu.VMEM((2 * BROWS + 8, 128), jnp.int32),
    pltpu.VMEM((128, 128), jnp.float32),
    pltpu.VMEM((128, 128), jnp.float32),
    pltpu.VMEM((128,), jnp.int32),
    pltpu.VMEM_SHARED((ACCR, 128), jnp.float32),
    pltpu.SemaphoreType.DMA,
    pltpu.SemaphoreType.DMA,
]


def _prefill_spare_row(src_v, dst_v):
    zero_i = jnp.zeros((16,), jnp.int32)
    dump_i = jnp.full((16,), DUMP, jnp.int32)
    row_i = jnp.full((16,), 2 * BROWS, jnp.int32)
    iota = lax.iota(jnp.int32, 16)
    for i in range(8):
        src_v[pl.ds(2 * BCAP + 16 * i, 16)] = zero_i
        plsc.store_scatter(dst_v, [row_i, iota + 16 * i], dump_i)

_IOTA16 = None


def _lane(v16, lane):
    iota = lax.iota(jnp.int32, 16)
    return jnp.sum(jnp.where(iota == lane, v16, 0), axis=0)


def _scat_ragged(table, src_v, dst_v, rows0, rows1, acc, sem0, sem1, n0, n1):
    ntot = n0 + n1

    def vrow(j):
        return jnp.where(j < n0, j, BROWS + (j - n0))

    def gather(j, buf, sem):
        idx = src_v.at[pl.ds(pl.multiple_of(vrow(j) * 128, 128), 128)]
        pltpu.async_copy(table.at[idx], buf, sem)

    def wait(buf, sem):
        pltpu.make_async_copy(
            table.at[src_v.at[pl.ds(0, 128)]], buf, sem
        ).wait()

    @pl.when(ntot > 0)
    def _():
        gather(0, rows0, sem0)

    def step(t, carry):
        j0 = 2 * t
        j1 = j0 + 1
        wait(rows0, sem0)
        gather(j1, rows1, sem1)
        pltpu.sync_copy(rows0, acc.at[dst_v.at[vrow(j0)]], add=True)
        wait(rows1, sem1)
        gather(j0 + 2, rows0, sem0)
        pltpu.sync_copy(rows1, acc.at[dst_v.at[vrow(j1)]], add=True)
        return carry

    npairs = (ntot + 1) >> 1

    @pl.when(npairs > 1)
    def _():
        lax.fori_loop(0, npairs - 1, step, 0)

    @pl.when(npairs > 0)
    def _():
        j0 = 2 * (npairs - 1)
        j1 = j0 + 1
        wait(rows0, sem0)
        gather(j1, rows1, sem1)
        pltpu.sync_copy(rows0, acc.at[dst_v.at[vrow(j0)]], add=True)
        wait(rows1, sem1)
        pltpu.sync_copy(rows1, acc.at[dst_v.at[vrow(j1)]], add=True)


def _load_slots(bsrc_sel, bdst, bcnt, k, s, src_v, dst_v, cnt_v):
    t0 = 2 * s
    t1 = 2 * s + 1
    pltpu.sync_copy(bsrc_sel.at[t0].at[k], src_v.at[pl.ds(0, BCAP)])
    pltpu.sync_copy(bsrc_sel.at[t1].at[k], src_v.at[pl.ds(BCAP, BCAP)])
    pltpu.sync_copy(bdst.at[t0].at[k], dst_v.at[pl.ds(0, BROWS)])
    pltpu.sync_copy(bdst.at[t1].at[k], dst_v.at[pl.ds(BROWS, BROWS)])
    pltpu.sync_copy(bcnt.at[t0], cnt_v)
    c0 = _lane(cnt_v[pl.ds(0, 16)], k)
    pltpu.sync_copy(bcnt.at[t1], cnt_v)
    c1 = _lane(cnt_v[pl.ds(0, 16)], k)
    return (c0 + 127) >> 7, (c1 + 127) >> 7


@functools.partial(
    pl.kernel,
    out_type=jax.ShapeDtypeStruct((NC, 2, ACCR, 128), jnp.float32),
    mesh=_mesh(),
    scratch_types=_SCAT_SCRATCH,
    compiler_params=pltpu.CompilerParams(needs_layout_passes=False),
)
def _scat_l1(table, bsrc, bsrcT, bdst, bcnt, out,
             src_v, dst_v, rows0, rows1, cnt_v, acc, sem0, sem1):
    c = lax.axis_index("c")
    s = lax.axis_index("s")
    base = TPT * s
    _prefill_spare_row(src_v, dst_v)

    for k in range(2):
        @pl.when(c == 0)
        def _():
            pltpu.sync_copy(bsrc.at[2 * s].at[k], src_v.at[pl.ds(0, BCAP)])
            pltpu.sync_copy(bsrc.at[2 * s + 1].at[k], src_v.at[pl.ds(BCAP, BCAP)])

        @pl.when(c == 1)
        def _():
            pltpu.sync_copy(bsrcT.at[2 * s].at[k], src_v.at[pl.ds(0, BCAP)])
            pltpu.sync_copy(bsrcT.at[2 * s + 1].at[k], src_v.at[pl.ds(BCAP, BCAP)])

        pltpu.sync_copy(bdst.at[2 * s].at[k], dst_v.at[pl.ds(0, BROWS)])
        pltpu.sync_copy(bdst.at[2 * s + 1].at[k], dst_v.at[pl.ds(BROWS, BROWS)])
        pltpu.sync_copy(bcnt.at[2 * s], cnt_v)
        c0 = _lane(cnt_v[pl.ds(0, 16)], k)
        pltpu.sync_copy(bcnt.at[2 * s + 1], cnt_v)
        c1 = _lane(cnt_v[pl.ds(0, 16)], k)
        n0 = (c0 + 127) >> 7
        n1 = (c1 + 127) >> 7

        pltpu.sync_copy(
            table.at[pl.ds(c * TPAD + PH * k + base, TPT)], acc.at[pl.ds(base, TPT)]
        )
        plsc.subcore_barrier()
        _scat_ragged(table, src_v, dst_v, rows0, rows1, acc, sem0, sem1, n0, n1)
        plsc.subcore_barrier()
        pltpu.sync_copy(
            acc.at[pl.ds(base, TPT)], out.at[c].at[k].at[pl.ds(base, TPT)]
        )


@functools.partial(
    pl.kernel,
    out_type=jax.ShapeDtypeStruct((NC, ACCR, 128), jnp.float32),
    mesh=_mesh(),
    scratch_types=_SCAT_SCRATCH,
    compiler_params=pltpu.CompilerParams(needs_layout_passes=False),
)
def _scat_l2(table, bsrc, bdst, bcnt, out,
             src_v, dst_v, rows0, rows1, cnt_v, acc, sem0, sem1):
    c = lax.axis_index("c")
    s = lax.axis_index("s")
    base = TPT * s
    _prefill_spare_row(src_v, dst_v)

    n0, n1 = _load_slots(bsrc, bdst, bcnt, c, s, src_v, dst_v, cnt_v)
    pltpu.sync_copy(table.at[pl.ds(PH * c + base, TPT)], acc.at[pl.ds(base, TPT)])
    plsc.subcore_barrier()
    _scat_ragged(table, src_v, dst_v, rows0, rows1, acc, sem0, sem1, n0, n1)
    plsc.subcore_barrier()
    pltpu.sync_copy(acc.at[pl.ds(base, TPT)], out.at[c].at[pl.ds(base, TPT)])


def _tc_layer1(x, W1, p):
    def f(x_ref, w_ref, p_ref, hs_ref, cnt_ref):
        h = jnp.dot(x_ref[...], w_ref[...], preferred_element_type=jnp.float32)
        cnt = jnp.sum(p_ref[...], axis=1, keepdims=True)
        cnt_ref[...] = cnt
        dinv = lax.rsqrt(cnt + 1.0)
        hp = h * dinv
        hs_ref[0] = hp[:, :128]
        hs_ref[1] = hp[:, 128:]

    return pl.pallas_call(
        f,
        grid=(N // R,),
        in_specs=[
            pl.BlockSpec((R, 128), lambda i: (i, 0)),
            pl.BlockSpec((128, 256), lambda i: (0, 0)),
            pl.BlockSpec((R, NW), lambda i: (i, 0)),
        ],
        out_specs=[
            pl.BlockSpec((2, R, 128), lambda i: (0, i, 0)),
            pl.BlockSpec((R, 1), lambda i: (i, 0)),
        ],
        out_shape=[
            jax.ShapeDtypeStruct((2, TPAD, 128), jnp.float32),
            jax.ShapeDtypeStruct((N, 1), jnp.float32),
        ],
    )(x, W1, p)


def _tc_layer2(s1, cnt, b1, W2):
    def f(s1_ref, cnt_ref, b1_ref, w_ref, g_ref):
        sm = jnp.concatenate([s1_ref[0], s1_ref[1]], axis=1)
        dinv = lax.rsqrt(cnt_ref[...] + 1.0)
        out1 = jnp.maximum(sm * dinv + b1_ref[...], 0.0)
        g = jnp.dot(out1, w_ref[...], preferred_element_type=jnp.float32)
        g_ref[...] = g * dinv

    return pl.pallas_call(
        f,
        grid=(N // R,),
        in_specs=[
            pl.BlockSpec((2, R, 128), lambda i: (0, i, 0)),
            pl.BlockSpec((R, 1), lambda i: (i, 0)),
            pl.BlockSpec((1, 256), lambda i: (0, 0)),
            pl.BlockSpec((256, 128), lambda i: (0, 0)),
        ],
        out_specs=pl.BlockSpec((R, 128), lambda i: (i, 0)),
        out_shape=jax.ShapeDtypeStruct((TPAD, 128), jnp.float32),
    )(s1, cnt, b1, W2)


def _tc_final(s2, cnt, b2, Wfc, bfc):
    def f(s2_ref, cnt_ref, b2_ref, w_ref, bfc_ref, o_ref):
        dinv = lax.rsqrt(cnt_ref[...] + 1.0)
        out2 = s2_ref[...] * dinv + b2_ref[...]
        o_ref[...] = (
            jnp.dot(out2, w_ref[...], preferred_element_type=jnp.float32)
            + bfc_ref[...]
        )

    return pl.pallas_call(
        f,
        grid=(N // R,),
        in_specs=[
            pl.BlockSpec((R, 128), lambda i: (i, 0)),
            pl.BlockSpec((R, 1), lambda i: (i, 0)),
            pl.BlockSpec((1, 128), lambda i: (0, 0)),
            pl.BlockSpec((128, 128), lambda i: (0, 0)),
            pl.BlockSpec((1, 128), lambda i: (0, 0)),
        ],
        out_specs=pl.BlockSpec((R, 128), lambda i: (i, 0)),
        out_shape=jax.ShapeDtypeStruct((N, 128), jnp.float32),
    )(s2, cnt, b2, Wfc, bfc)


def _merge_phases(s):
    return jnp.concatenate([s[..., 0, :PH, :], s[..., 1, : N - PH, :]], axis=-2)


def kernel(x, edge_index, W1, b1, W2, b2, Wfc, bfc):
    src = edge_index[0].astype(jnp.int32)
    dst = edge_index[1].astype(jnp.int32)

    bsrc, bsrcT, bdst, bcnt, bhist = _bucket_kernel(src, dst)
    p = bhist.T[:N]

    hs, cnt = _tc_layer1(x, W1, p)
    s1 = _scat_l1(hs.reshape(2 * TPAD, 128), bsrc, bsrcT, bdst, bcnt)
    gs = _tc_layer2(_merge_phases(s1), cnt, b1.reshape(1, 256), W2)
    s2 = _scat_l2(gs, bsrc, bdst, bcnt)
    out = _tc_final(
        _merge_phases(s2), cnt, b2.reshape(1, 128), Wfc, bfc.reshape(1, 128)
    )
    return out


# speedup vs baseline: 1.1212x; 1.1212x over previous
import functools

import jax
import jax.numpy as jnp
from jax import lax
from jax.experimental import pallas as pl
from jax.experimental.pallas import tpu as pltpu
from jax.experimental.pallas import tpu_sc as plsc

N = 10000
E = 320000
NC = 2
NT = 16
NW = NC * NT
R = 1000

PH = 5120
ACCR = 5248
TPT = ACCR // NT
DUMP = 5200
TPAD = 10496

NP = 10240
ED_MAIN = 10112
ED_LAST = E - (NW - 1) * ED_MAIN
BCAP = 10240
BROWS = BCAP // 128


def _mesh():
    return plsc.VectorSubcoreMesh(
        core_axis_name="c", subcore_axis_name="s", num_cores=NC, num_subcores=NT
    )


_BUCKET_KW = dict(
    out_type=[
        jax.ShapeDtypeStruct((NW, 2, BCAP), jnp.int32),
        jax.ShapeDtypeStruct((NW, 2, BCAP), jnp.int32),
        jax.ShapeDtypeStruct((NW, 2, BROWS, 128), jnp.int32),
        jax.ShapeDtypeStruct((NW, 128), jnp.int32),
        jax.ShapeDtypeStruct((NW, NP), jnp.float32),
    ],
    scratch_types=[
        pltpu.VMEM((ED_MAIN,), jnp.int32),
        pltpu.VMEM((ED_MAIN,), jnp.int32),
        pltpu.VMEM((NP,), jnp.float32),
        pltpu.VMEM((BCAP,), jnp.int32),
        pltpu.VMEM((BCAP,), jnp.int32),
        pltpu.VMEM((BCAP,), jnp.int32),
        pltpu.VMEM((BCAP,), jnp.int32),
        pltpu.VMEM((BROWS, 128), jnp.int32),
        pltpu.VMEM((BROWS, 128), jnp.int32),
        pltpu.VMEM((128,), jnp.int32),
    ],
    compiler_params=pltpu.CompilerParams(needs_layout_passes=False),
)


def _bucket_body(
    srcflat, dstflat, bsrc, bsrcT, bdst, bcnt, bhist,
    in_s, in_d, hist, b0s, b1s, b0t, b1t, b0d, b1d, cnt_v
):
    c = lax.axis_index("c")
    s = lax.axis_index("s")
    w = c * NT + s
    zeros_f = jnp.zeros((16,), jnp.float32)
    zero_i = jnp.zeros((16,), jnp.int32)
    ones_f = jnp.full((16,), 1.0, jnp.float32)
    dump_i = jnp.full((16,), DUMP, jnp.int32)
    iota = lax.iota(jnp.int32, 16)

    def zstep(i, carry):
        off = pl.ds(pl.multiple_of(16 * i, 16), 16)
        hist[off] = zeros_f
        return carry

    lax.fori_loop(0, NP // 16, zstep, 0)

    def pstep(i, carry):
        off = pl.ds(pl.multiple_of(16 * i, 16), 16)
        b0s[off] = zero_i
        b1s[off] = zero_i
        b0t[off] = zero_i + TPAD
        b1t[off] = zero_i + TPAD
        pos = iota + 16 * i
        rowi = pos >> 7
        coli = pos & 127
        plsc.store_scatter(b0d, [rowi, coli], dump_i)
        plsc.store_scatter(b1d, [rowi, coli], dump_i)
        return carry

    lax.fori_loop(0, BCAP // 16, pstep, 0)

    @pl.when(w < NW - 1)
    def _():
        pltpu.sync_copy(srcflat.at[pl.ds(ED_MAIN * w, ED_MAIN)], in_s)
        pltpu.sync_copy(dstflat.at[pl.ds(ED_MAIN * w, ED_MAIN)], in_d)

    @pl.when(w == NW - 1)
    def _():
        pltpu.sync_copy(
            srcflat.at[pl.ds(ED_MAIN * w, ED_LAST)], in_s.at[pl.ds(0, ED_LAST)]
        )
        pltpu.sync_copy(
            dstflat.at[pl.ds(ED_MAIN * w, ED_LAST)], in_d.at[pl.ds(0, ED_LAST)]
        )

    nvec = jnp.where(w == NW - 1, ED_LAST // 16, ED_MAIN // 16)

    def step(i, carry):
        off0, off1 = carry
        sl = pl.ds(pl.multiple_of(16 * i, 16), 16)
        sv = in_s[sl]
        dv = in_d[sl]
        plsc.addupdate_scatter(hist, [dv], ones_f)
        m0 = dv < PH
        m1 = jnp.logical_not(m0)
        r0 = plsc.cumsum(m0.astype(jnp.int32))
        r1 = plsc.cumsum(m1.astype(jnp.int32))
        pos0 = off0 + r0 - 1
        pos1 = off1 + r1 - 1
        plsc.store_scatter(b0s, [pos0], sv, mask=m0)
        plsc.store_scatter(b1s, [pos1], sv, mask=m1)
        plsc.store_scatter(b0t, [pos0], sv + TPAD, mask=m0)
        plsc.store_scatter(b1t, [pos1], sv + TPAD, mask=m1)
        plsc.store_scatter(b0d, [pos0 >> 7, pos0 & 127], dv, mask=m0)
        plsc.store_scatter(b1d, [pos1 >> 7, pos1 & 127], dv - PH, mask=m1)
        return (off0 + jnp.max(r0), off1 + jnp.max(r1))

    off0, off1 = lax.fori_loop(0, nvec, step, (jnp.int32(0), jnp.int32(0)))

    for i in range(8):
        cnt_v[pl.ds(16 * i, 16)] = zero_i
    cnt_v[pl.ds(0, 16)] = off0 * (iota == 0) + off1 * (iota == 1)

    pltpu.sync_copy(hist, bhist.at[w])
    pltpu.sync_copy(cnt_v, bcnt.at[w])
    pltpu.sync_copy(b0s, bsrc.at[w].at[0])
    pltpu.sync_copy(b1s, bsrc.at[w].at[1])
    pltpu.sync_copy(b0t, bsrcT.at[w].at[0])
    pltpu.sync_copy(b1t, bsrcT.at[w].at[1])
    pltpu.sync_copy(b0d, bdst.at[w].at[0])
    pltpu.sync_copy(b1d, bdst.at[w].at[1])


@functools.lru_cache(maxsize=None)
def _built(which):
    if which == "bucket":
        return pl.kernel(_bucket_body, mesh=_mesh(), **_BUCKET_KW)
    kw = dict(
        mesh=_mesh(),
        scratch_types=tuple(_SCAT_SCRATCH),
        compiler_params=pltpu.CompilerParams(needs_layout_passes=False),
    )
    if which == "l1":
        return pl.kernel(
            _scat_l1_body,
            out_type=jax.ShapeDtypeStruct((NC, 2, ACCR, 128), jnp.float32),
            **kw,
        )
    return pl.kernel(
        _scat_l2_body,
        out_type=jax.ShapeDtypeStruct((NC, ACCR, 128), jnp.float32),
        **kw,
    )


def _bucket_kernel(*args):
    return _built("bucket")(*args)


def _scat_l1(*args):
    return _built("l1")(*args)


def _scat_l2(*args):
    return _built("l2")(*args)


_SCAT_SCRATCH = [
    pltpu.VMEM((2 * BCAP,), jnp.int32),
    pltpu.VMEM((2 * BROWS, 128), jnp.int32),
    pltpu.VMEM((128, 128), jnp.float32),
    pltpu.VMEM((128, 128), jnp.float32),
    pltpu.VMEM((128,), jnp.int32),
    pltpu.VMEM_SHARED((ACCR, 128), jnp.float32),
    pltpu.SemaphoreType.DMA,
    pltpu.SemaphoreType.DMA,
]

_IOTA16 = None


def _lane(v16, lane):
    iota = lax.iota(jnp.int32, 16)
    return jnp.sum(jnp.where(iota == lane, v16, 0), axis=0)


def _scat_ragged(table, src_v, dst_v, rows0, rows1, acc, sem0, sem1, n0, n1):
    ntot = n0 + n1

    def vrow(j):
        return jnp.where(j < n0, j, BROWS + (j - n0))

    def gather(j, buf, sem):
        idx = src_v.at[pl.ds(pl.multiple_of(vrow(j) * 128, 128), 128)]
        pltpu.async_copy(table.at[idx], buf, sem)

    def wait(buf, sem):
        pltpu.make_async_copy(
            table.at[src_v.at[pl.ds(0, 128)]], buf, sem
        ).wait()

    @pl.when(ntot > 0)
    def _():
        gather(0, rows0, sem0)

    def step(t, carry):
        j0 = 2 * t
        j1 = j0 + 1
        wait(rows0, sem0)

        @pl.when(j1 < ntot)
        def _():
            gather(j1, rows1, sem1)

        pltpu.sync_copy(rows0, acc.at[dst_v.at[vrow(j0)]], add=True)

        @pl.when(j1 < ntot)
        def _():
            wait(rows1, sem1)

            @pl.when(j0 + 2 < ntot)
            def _():
                gather(j0 + 2, rows0, sem0)

            pltpu.sync_copy(rows1, acc.at[dst_v.at[vrow(j1)]], add=True)

        return carry

    lax.fori_loop(0, (ntot + 1) // 2, step, 0)


def _load_slots(bsrc_sel, bdst, bcnt, k, s, src_v, dst_v, cnt_v):
    t0 = 2 * s
    t1 = 2 * s + 1
    pltpu.sync_copy(bsrc_sel.at[t0].at[k], src_v.at[pl.ds(0, BCAP)])
    pltpu.sync_copy(bsrc_sel.at[t1].at[k], src_v.at[pl.ds(BCAP, BCAP)])
    pltpu.sync_copy(bdst.at[t0].at[k], dst_v.at[pl.ds(0, BROWS)])
    pltpu.sync_copy(bdst.at[t1].at[k], dst_v.at[pl.ds(BROWS, BROWS)])
    pltpu.sync_copy(bcnt.at[t0], cnt_v)
    c0 = _lane(cnt_v[pl.ds(0, 16)], k)
    pltpu.sync_copy(bcnt.at[t1], cnt_v)
    c1 = _lane(cnt_v[pl.ds(0, 16)], k)
    return (c0 + 127) >> 7, (c1 + 127) >> 7


def _scat_l1_body(table, bsrc, bsrcT, bdst, bcnt, out,
                  src_v, dst_v, rows0, rows1, cnt_v, acc, sem0, sem1):
    c = lax.axis_index("c")
    s = lax.axis_index("s")
    base = TPT * s

    for k in range(2):
        @pl.when(c == 0)
        def _():
            pltpu.sync_copy(bsrc.at[2 * s].at[k], src_v.at[pl.ds(0, BCAP)])
            pltpu.sync_copy(bsrc.at[2 * s + 1].at[k], src_v.at[pl.ds(BCAP, BCAP)])

        @pl.when(c == 1)
        def _():
            pltpu.sync_copy(bsrcT.at[2 * s].at[k], src_v.at[pl.ds(0, BCAP)])
            pltpu.sync_copy(bsrcT.at[2 * s + 1].at[k], src_v.at[pl.ds(BCAP, BCAP)])

        pltpu.sync_copy(bdst.at[2 * s].at[k], dst_v.at[pl.ds(0, BROWS)])
        pltpu.sync_copy(bdst.at[2 * s + 1].at[k], dst_v.at[pl.ds(BROWS, BROWS)])
        pltpu.sync_copy(bcnt.at[2 * s], cnt_v)
        c0 = _lane(cnt_v[pl.ds(0, 16)], k)
        pltpu.sync_copy(bcnt.at[2 * s + 1], cnt_v)
        c1 = _lane(cnt_v[pl.ds(0, 16)], k)
        n0 = (c0 + 127) >> 7
        n1 = (c1 + 127) >> 7

        pltpu.sync_copy(
            table.at[pl.ds(c * TPAD + PH * k + base, TPT)], acc.at[pl.ds(base, TPT)]
        )
        plsc.subcore_barrier()
        _scat_ragged(table, src_v, dst_v, rows0, rows1, acc, sem0, sem1, n0, n1)
        plsc.subcore_barrier()
        pltpu.sync_copy(
            acc.at[pl.ds(base, TPT)], out.at[c].at[k].at[pl.ds(base, TPT)]
        )


def _scat_l2_body(table, bsrc, bdst, bcnt, out,
                  src_v, dst_v, rows0, rows1, cnt_v, acc, sem0, sem1):
    c = lax.axis_index("c")
    s = lax.axis_index("s")
    base = TPT * s

    n0, n1 = _load_slots(bsrc, bdst, bcnt, c, s, src_v, dst_v, cnt_v)
    pltpu.sync_copy(table.at[pl.ds(PH * c + base, TPT)], acc.at[pl.ds(base, TPT)])
    plsc.subcore_barrier()
    _scat_ragged(table, src_v, dst_v, rows0, rows1, acc, sem0, sem1, n0, n1)
    plsc.subcore_barrier()
    pltpu.sync_copy(acc.at[pl.ds(base, TPT)], out.at[c].at[pl.ds(base, TPT)])


def _tc_layer1(x, W1, p):
    def f(x_ref, w_ref, p_ref, hs_ref, cnt_ref):
        h = jnp.dot(x_ref[...], w_ref[...], preferred_element_type=jnp.float32)
        cnt = jnp.sum(p_ref[...], axis=1, keepdims=True)
        cnt_ref[...] = cnt
        dinv = lax.rsqrt(cnt + 1.0)
        hp = h * dinv
        hs_ref[0] = hp[:, :128]
        hs_ref[1] = hp[:, 128:]

    return pl.pallas_call(
        f,
        grid=(N // R,),
        in_specs=[
            pl.BlockSpec((R, 128), lambda i: (i, 0)),
            pl.BlockSpec((128, 256), lambda i: (0, 0)),
            pl.BlockSpec((R, NW), lambda i: (i, 0)),
        ],
        out_specs=[
            pl.BlockSpec((2, R, 128), lambda i: (0, i, 0)),
            pl.BlockSpec((R, 1), lambda i: (i, 0)),
        ],
        out_shape=[
            jax.ShapeDtypeStruct((2, TPAD, 128), jnp.float32),
            jax.ShapeDtypeStruct((N, 1), jnp.float32),
        ],
    )(x, W1, p)


def _tc_layer2(s1, cnt, b1, W2):
    def f(s1_ref, cnt_ref, b1_ref, w_ref, g_ref):
        sm = jnp.concatenate([s1_ref[0], s1_ref[1]], axis=1)
        dinv = lax.rsqrt(cnt_ref[...] + 1.0)
        out1 = jnp.maximum(sm * dinv + b1_ref[...], 0.0)
        g = jnp.dot(out1, w_ref[...], preferred_element_type=jnp.float32)
        g_ref[...] = g * dinv

    return pl.pallas_call(
        f,
        grid=(N // R,),
        in_specs=[
            pl.BlockSpec((2, R, 128), lambda i: (0, i, 0)),
            pl.BlockSpec((R, 1), lambda i: (i, 0)),
            pl.BlockSpec((1, 256), lambda i: (0, 0)),
            pl.BlockSpec((256, 128), lambda i: (0, 0)),
        ],
        out_specs=pl.BlockSpec((R, 128), lambda i: (i, 0)),
        out_shape=jax.ShapeDtypeStruct((TPAD, 128), jnp.float32),
    )(s1, cnt, b1, W2)


def _tc_final(s2, cnt, b2, Wfc, bfc):
    def f(s2_ref, cnt_ref, b2_ref, w_ref, bfc_ref, o_ref):
        dinv = lax.rsqrt(cnt_ref[...] + 1.0)
        out2 = s2_ref[...] * dinv + b2_ref[...]
        o_ref[...] = (
            jnp.dot(out2, w_ref[...], preferred_element_type=jnp.float32)
            + bfc_ref[...]
        )

    return pl.pallas_call(
        f,
        grid=(N // R,),
        in_specs=[
            pl.BlockSpec((R, 128), lambda i: (i, 0)),
            pl.BlockSpec((R, 1), lambda i: (i, 0)),
            pl.BlockSpec((1, 128), lambda i: (0, 0)),
            pl.BlockSpec((128, 128), lambda i: (0, 0)),
            pl.BlockSpec((1, 128), lambda i: (0, 0)),
        ],
        out_specs=pl.BlockSpec((R, 128), lambda i: (i, 0)),
        out_shape=jax.ShapeDtypeStruct((N, 128), jnp.float32),
    )(s2, cnt, b2, Wfc, bfc)


def _merge_phases(s):
    return jnp.concatenate([s[..., 0, :PH, :], s[..., 1, : N - PH, :]], axis=-2)


def kernel(x, edge_index, W1, b1, W2, b2, Wfc, bfc):
    src = edge_index[0].astype(jnp.int32)
    dst = edge_index[1].astype(jnp.int32)

    bsrc, bsrcT, bdst, bcnt, bhist = _bucket_kernel(src, dst)
    p = bhist.T[:N]

    hs, cnt = _tc_layer1(x, W1, p)
    s1 = _scat_l1(hs.reshape(2 * TPAD, 128), bsrc, bsrcT, bdst, bcnt)
    gs = _tc_layer2(_merge_phases(s1), cnt, b1.reshape(1, 256), W2)
    s2 = _scat_l2(gs, bsrc, bdst, bcnt)
    out = _tc_final(
        _merge_phases(s2), cnt, b2.reshape(1, 128), Wfc, bfc.reshape(1, 128)
    )
    return out
